# Initial kernel scaffold; baseline (speedup 1.0000x reference)
import jax
import jax.numpy as jnp
from jax.experimental import pallas as pl


def kernel(rots, trans, node_features, batch, x_mask, noising_mask, sampled_edge_index, seq_local_edge_index, W_edge, W_alpha, a_vec, W_v, W_o, W_ffn1, W_ffn2, W_rot1, b_rot1, W_rot2, b_rot2, W_rot3, b_rot3, W_t):
    raise NotImplementedError("write your pallas kernel here")


# speedup vs baseline: 6.0031x; 6.0031x over previous
import functools

import jax
import jax.numpy as jnp
from jax import lax
from jax.experimental import pallas as pl
from jax.experimental.pallas import tpu as pltpu
from jax.experimental.pallas import tpu_sc as plsc

N = 10000
H = 32
FC = 3
SPH = 9
HEADS = 8
ACH = 16
VC = 8
D = H + FC
E = 140000
EB = 64
NBLK = 70
NW = 32
EP = NW * NBLK * EB
NROWS = EP // EB
CH = 144
NCHUNK = 4
BN = 400
BE = 512

_IDEAL = ((-0.525, 1.363, 0.0), (0.0, 0.0, 0.0), (1.526, 0.0, 0.0))
import numpy as _np
_MU = _np.linspace(0.0, 20.0, 16).astype(_np.float32)
_FREQ = _np.exp(_np.arange(0, 16, 2).astype(_np.float32)
                * (-_np.log(10000.0) / 16.0)).astype(_np.float32)


def _k1_body(nf_r, rots9_r, trans_r, nm_r, xm_r, wad, was, wv,
             vt0, vt1, vt2, vt3, pd, ps, t16):
    nf = nf_r[...]
    rots9 = rots9_r[...]
    trans = trans_r[...]
    b = nf.shape[0]
    ed = nm_r[...] * (1.0 - xm_r[...])
    zc = jnp.zeros((b, 1), jnp.float32)
    row0 = jnp.concatenate([zc, zc, ed], axis=1)[:, None, :]
    rows = [row0]
    for i in range(3):
        r0 = rots9[:, 3 * i + 0]
        r1 = rots9[:, 3 * i + 1]
        ti = trans[:, i]
        fa0 = _IDEAL[0][0] * r0 + _IDEAL[0][1] * r1 + ti
        fa1 = ti
        fa2 = _IDEAL[2][0] * r0 + ti
        rows.append(jnp.stack([fa0, fa1, fa2], axis=1)[:, None, :])
    rows.append(jnp.zeros((b, 5, 3), jnp.float32))
    extra = jnp.concatenate(rows, axis=1)
    fused = jnp.concatenate([nf, extra], axis=2)
    x0 = fused[:, 0, :]
    pd[...] = jnp.dot(x0, wad[...], preferred_element_type=jnp.float32)
    ps[...] = jnp.dot(x0, was[...], preferred_element_type=jnp.float32)
    v = jnp.dot(fused.reshape(b * SPH, D), wv[...],
                preferred_element_type=jnp.float32).reshape(b, SPH, HEADS * VC)
    vt0[...] = jnp.concatenate([v[:, 0, :], v[:, 1, :], v[:, 2, :16]], axis=1)
    vt1[...] = jnp.concatenate([v[:, 2, 16:], v[:, 3, :], v[:, 4, :32]], axis=1)
    vt2[...] = jnp.concatenate([v[:, 4, 32:], v[:, 5, :], v[:, 6, :48]], axis=1)
    vt3[...] = jnp.concatenate([v[:, 6, 48:], v[:, 7, :], v[:, 8, :]], axis=1)
    t16[...] = jnp.concatenate([trans, jnp.zeros((b, 13), jnp.float32)], axis=1)


def _k1(nf, rots9, trans, nm, xm, wad, was, wv):
    nb = N // BN
    fsd = jax.ShapeDtypeStruct
    return pl.pallas_call(
        _k1_body,
        grid=(nb,),
        in_specs=[
            pl.BlockSpec((BN, SPH, H), lambda i: (i, 0, 0)),
            pl.BlockSpec((BN, 9), lambda i: (i, 0)),
            pl.BlockSpec((BN, 3), lambda i: (i, 0)),
            pl.BlockSpec((BN, 1), lambda i: (i, 0)),
            pl.BlockSpec((BN, 1), lambda i: (i, 0)),
            pl.BlockSpec((D, 128), lambda i: (0, 0)),
            pl.BlockSpec((D, 128), lambda i: (0, 0)),
            pl.BlockSpec((D, HEADS * VC), lambda i: (0, 0)),
        ],
        out_specs=[
            pl.BlockSpec((BN, CH), lambda i: (i, 0)),
            pl.BlockSpec((BN, CH), lambda i: (i, 0)),
            pl.BlockSpec((BN, CH), lambda i: (i, 0)),
            pl.BlockSpec((BN, CH), lambda i: (i, 0)),
            pl.BlockSpec((BN, 128), lambda i: (i, 0)),
            pl.BlockSpec((BN, 128), lambda i: (i, 0)),
            pl.BlockSpec((BN, 16), lambda i: (i, 0)),
        ],
        out_shape=[
            fsd((N, CH), jnp.float32), fsd((N, CH), jnp.float32),
            fsd((N, CH), jnp.float32), fsd((N, CH), jnp.float32),
            fsd((N, 128), jnp.float32), fsd((N, 128), jnp.float32),
            fsd((N, 16), jnp.float32),
        ],
    )(nf, rots9, trans, nm, xm, wad, was, wv)


def _k2_body(pd_hbm, ps_hbm, t16_hbm, src_hbm, dst_hbm,
             psum_hbm, vecd_hbm, dvec_hbm,
             idx_s, idx_d, a, bb, ts, td, dfbuf, sem):
    c = lax.axis_index("c")
    s = lax.axis_index("s")
    wid = c * 16 + s

    def block(j, carry):
        row = wid * NBLK + j
        pltpu.sync_copy(src_hbm.at[row], idx_s)
        pltpu.sync_copy(dst_hbm.at[row], idx_d)
        pltpu.async_copy(pd_hbm.at[idx_d], a, sem).wait()
        pltpu.async_copy(ps_hbm.at[idx_s], bb, sem).wait()
        pltpu.async_copy(t16_hbm.at[idx_s], ts, sem).wait()
        pltpu.async_copy(t16_hbm.at[idx_d], td, sem).wait()

        def edge(i, carry2):
            for k in range(8):
                sl = pl.ds(k * 16, 16)
                a[i, sl] = a[i, sl] + bb[i, sl]
            ts[i, :] = ts[i, :] - td[i, :]
            return carry2

        lax.fori_loop(0, EB, edge, 0)
        for t in range(EB // 16):
            sv = idx_s[pl.ds(t * 16, 16)]
            dv = idx_d[pl.ds(t * 16, 16)]
            dfbuf[pl.ds(t * 16, 16)] = (sv - dv).astype(jnp.float32)
        pltpu.sync_copy(a, psum_hbm.at[pl.ds(row * EB, EB)])
        pltpu.sync_copy(ts, vecd_hbm.at[pl.ds(row * EB, EB)])
        pltpu.sync_copy(dfbuf, dvec_hbm.at[pl.ds(row * EB, EB)])
        return carry

    lax.fori_loop(0, NBLK, block, 0)


def _k2(pd, ps, t16, src2d, dst2d):
    fsd = jax.ShapeDtypeStruct
    kern = functools.partial(
        pl.kernel,
        out_type=[fsd((EP, 128), jnp.float32), fsd((EP, 16), jnp.float32),
                  fsd((EP,), jnp.float32)],
        mesh=plsc.VectorSubcoreMesh(core_axis_name="c", subcore_axis_name="s"),
        compiler_params=pltpu.CompilerParams(use_tc_tiling_on_sc=False),
        scratch_types=[
            pltpu.VMEM((EB,), jnp.int32),
            pltpu.VMEM((EB,), jnp.int32),
            pltpu.VMEM((EB, 128), jnp.float32),
            pltpu.VMEM((EB, 128), jnp.float32),
            pltpu.VMEM((EB, 16), jnp.float32),
            pltpu.VMEM((EB, 16), jnp.float32),
            pltpu.VMEM((EB,), jnp.float32),
            pltpu.SemaphoreType.DMA,
        ],
    )(_k2_body)
    return kern(pd, ps, t16, src2d, dst2d)


def _k3_body(psum, vecd_r, dfl_r, wedge, wae, avec, mu_r, fr_r, w16):
    b = psum.shape[0]
    vecd = vecd_r[...]
    vec = vecd[:, 0:3]
    dist = jnp.sqrt(jnp.sum(vec * vec, axis=1))
    dfloat = dfl_r[...]
    mu = mu_r[...][0]
    sigma = 20.0 / 16.0
    rbf = jnp.exp(-(((dist[:, None] - mu[None, :]) / sigma) ** 2))
    freq = fr_r[...][0]
    ang = dfloat[:, None] * freq[None, :]
    pe = jnp.concatenate([jnp.cos(ang), jnp.sin(ang)], axis=1)
    ef = jnp.concatenate([rbf, pe], axis=1)
    ee = jax.nn.relu(jnp.dot(ef, wedge[...],
                             preferred_element_type=jnp.float32))
    h = psum[...] + jnp.dot(ee, wae[...],
                            preferred_element_type=jnp.float32)
    h = jax.nn.leaky_relu(h)
    ha = h * avec[...]
    alpha = jnp.concatenate(
        [jnp.sum(ha[:, ACH * hd:ACH * (hd + 1)], axis=1, keepdims=True)
         for hd in range(HEADS)], axis=1)
    valid = jnp.isfinite(dist) & (dist > 1e-3)
    w = jnp.where(valid[:, None], jnp.exp(alpha), 0.0)
    w16[...] = jnp.concatenate([w, jnp.zeros((b, 8), jnp.float32)], axis=1)


def _k3(psum, vecd, dvec, wedge, wae, avec):
    nb = EP // BE
    mu = jnp.asarray(_MU, jnp.float32).reshape(1, 16)
    fr = jnp.asarray(_FREQ, jnp.float32).reshape(1, 8)
    return pl.pallas_call(
        _k3_body,
        grid=(nb,),
        in_specs=[
            pl.BlockSpec((BE, 128), lambda i: (i, 0)),
            pl.BlockSpec((BE, 16), lambda i: (i, 0)),
            pl.BlockSpec((BE,), lambda i: (i,)),
            pl.BlockSpec((H, H), lambda i: (0, 0)),
            pl.BlockSpec((H, 128), lambda i: (0, 0)),
            pl.BlockSpec((1, HEADS * ACH), lambda i: (0, 0)),
            pl.BlockSpec((1, 16), lambda i: (0, 0)),
            pl.BlockSpec((1, 8), lambda i: (0, 0)),
        ],
        out_specs=[pl.BlockSpec((BE, 16), lambda i: (i, 0))],
        out_shape=[jax.ShapeDtypeStruct((EP, 16), jnp.float32)],
    )(psum, vecd, dvec, wedge, wae, avec.reshape(1, HEADS * ACH), mu, fr)


def _k4_body(vt0, vt1, vt2, vt3, w16_hbm, src_hbm, dst_hbm, z144, z16,
             a0, a1, a2, a3, den,
             idx_s, idx_d, vrows, wrows, acc, accd, sem):
    c = lax.axis_index("c")
    s = lax.axis_index("s")
    wid = c * 16 + s
    rows_per_tile = N // 16
    vts = [vt0, vt1, vt2, vt3]
    outs = [a0, a1, a2, a3]
    lanes = lax.iota(jnp.int32, 16)

    for chunk in range(NCHUNK):
        pltpu.sync_copy(z144.at[pl.ds(s * rows_per_tile, rows_per_tile)],
                        acc.at[pl.ds(s * rows_per_tile, rows_per_tile)])
        if chunk == 0:
            pltpu.sync_copy(z16.at[pl.ds(s * rows_per_tile, rows_per_tile)],
                            accd.at[pl.ds(s * rows_per_tile, rows_per_tile)])
        plsc.subcore_barrier()

        def block(j, carry):
            row = wid * NBLK + j
            pltpu.sync_copy(src_hbm.at[row], idx_s)
            pltpu.sync_copy(dst_hbm.at[row], idx_d)
            pltpu.async_copy(vts[chunk].at[idx_s], vrows, sem).wait()
            pltpu.sync_copy(w16_hbm.at[pl.ds(row * EB, EB)], wrows)

            def edge(i, carry2):
                wr = wrows[i, :]
                wvs = [jnp.where(lanes < 8, wr[2 * p], wr[2 * p + 1])
                       for p in range(4)]
                for jj in range(CH // 16):
                    p = (9 * chunk + jj) % 4
                    sl = pl.ds(jj * 16, 16)
                    vrows[i, sl] = vrows[i, sl] * wvs[p]
                return carry2

            lax.fori_loop(0, EB, edge, 0)
            pltpu.sync_copy(vrows, acc.at[idx_d], add=True)
            if chunk == 0:
                pltpu.sync_copy(wrows, accd.at[idx_d], add=True)
            return carry

        lax.fori_loop(0, NBLK, block, 0)
        plsc.subcore_barrier()
        pltpu.sync_copy(
            acc.at[pl.ds(s * rows_per_tile, rows_per_tile)],
            outs[chunk].at[pl.ds(c * N + s * rows_per_tile, rows_per_tile)])
        if chunk == 0:
            pltpu.sync_copy(
                accd.at[pl.ds(s * rows_per_tile, rows_per_tile)],
                den.at[pl.ds(c * N + s * rows_per_tile, rows_per_tile)])
        plsc.subcore_barrier()


def _k4(vt0, vt1, vt2, vt3, w16, src2d, dst2d, z144, z16):
    fsd = jax.ShapeDtypeStruct
    kern = functools.partial(
        pl.kernel,
        out_type=[
            fsd((2 * N, CH), jnp.float32), fsd((2 * N, CH), jnp.float32),
            fsd((2 * N, CH), jnp.float32), fsd((2 * N, CH), jnp.float32),
            fsd((2 * N, 16), jnp.float32),
        ],
        mesh=plsc.VectorSubcoreMesh(core_axis_name="c", subcore_axis_name="s"),
        compiler_params=pltpu.CompilerParams(use_tc_tiling_on_sc=False),
        scratch_types=[
            pltpu.VMEM((EB,), jnp.int32),
            pltpu.VMEM((EB,), jnp.int32),
            pltpu.VMEM((EB, CH), jnp.float32),
            pltpu.VMEM((EB, 16), jnp.float32),
            pltpu.VMEM_SHARED((N, CH), jnp.float32),
            pltpu.VMEM_SHARED((N, 16), jnp.float32),
            pltpu.SemaphoreType.DMA,
        ],
    )(_k4_body)
    return kern(vt0, vt1, vt2, vt3, w16, src2d, dst2d, z144, z16)


def _k5_body(a0a, a0b, a1a, a1b, a2a, a2b, a3a, a3b, dena, denb,
             rots9, trans, nm, wo, wffn1, wffn2,
             wr1, br1, wr2, br2, wr3, br3, wt,
             rots_out, trans_out, bb_out):
    b = rots9.shape[0]
    den = dena[...] + denb[...]
    den8 = den[:, :HEADS] + 1e-9
    chunks = []
    for (pa, pb, k) in ((a0a, a0b, 0), (a1a, a1b, 1),
                        (a2a, a2b, 2), (a3a, a3b, 3)):
        ck = pa[...] + pb[...]
        dpieces = [jnp.broadcast_to(den8[:, (18 * k + t) % 8][:, None],
                                    (b, 8)) for t in range(18)]
        ck = ck / jnp.concatenate(dpieces, axis=1)
        chunks.append(ck)
    c0, c1, c2, c3 = chunks
    planes = [
        c0[:, 0:64], c0[:, 64:128],
        jnp.concatenate([c0[:, 128:144], c1[:, 0:48]], axis=1),
        c1[:, 48:112],
        jnp.concatenate([c1[:, 112:144], c2[:, 0:32]], axis=1),
        c2[:, 32:96],
        jnp.concatenate([c2[:, 96:144], c3[:, 0:16]], axis=1),
        c3[:, 16:80], c3[:, 80:144],
    ]
    agg = jnp.concatenate([p[:, None, :] for p in planes], axis=1)
    out = jnp.dot(agg.reshape(b * SPH, HEADS * VC), wo[...],
                  preferred_element_type=jnp.float32).reshape(b, SPH, H)
    ff = jnp.dot(out.reshape(b * SPH, H), wffn1[...],
                 preferred_element_type=jnp.float32).reshape(b, SPH, H)
    ff0 = jax.nn.gelu(ff[:, 0, :])
    ff = jnp.concatenate([ff0[:, None, :], ff[:, 1:, :]], axis=1)
    bb = out + jnp.dot(ff.reshape(b * SPH, H), wffn2[...],
                       preferred_element_type=jnp.float32).reshape(b, SPH, H)
    bb_out[...] = bb
    inv = bb[:, 0, :]
    h1 = jax.nn.relu(jnp.dot(inv, wr1[...],
                             preferred_element_type=jnp.float32) + br1[...])
    h2 = jax.nn.relu(jnp.dot(h1, wr2[...],
                             preferred_element_type=jnp.float32) + br2[...])
    qv = jnp.dot(h2, wr3[...], preferred_element_type=jnp.float32) + br3[...]
    qx, qy, qz = qv[:, 0], qv[:, 1], qv[:, 2]
    qn = jnp.sqrt(1.0 + qx * qx + qy * qy + qz * qz)
    w_, x_, y_, z_ = 1.0 / qn, qx / qn, qy / qn, qz / qn
    u = [1 - 2 * (y_ * y_ + z_ * z_), 2 * (x_ * y_ - w_ * z_), 2 * (x_ * z_ + w_ * y_),
         2 * (x_ * y_ + w_ * z_), 1 - 2 * (x_ * x_ + z_ * z_), 2 * (y_ * z_ - w_ * x_),
         2 * (x_ * z_ - w_ * y_), 2 * (y_ * z_ + w_ * x_), 1 - 2 * (x_ * x_ + y_ * y_)]
    tupd = jnp.sum(bb[:, 1:4, :] * wt[...].reshape(1, 1, H), axis=2)
    r = rots9[...]
    del rots9
    ru = []
    for i in range(3):
        for kk in range(3):
            ru.append(r[:, 3 * i + 0] * u[0 + kk] + r[:, 3 * i + 1] * u[3 + kk]
                      + r[:, 3 * i + 2] * u[6 + kk])
    rotsu = jnp.stack(ru, axis=1)
    mk = nm[...]
    rots_out[...] = mk * rotsu + (1.0 - mk) * r
    trans_out[...] = trans[...] + mk * tupd


def _k5(a0, a1, a2, a3, den, rots9, trans, nm, wo, wffn1, wffn2,
        wr1, br1, wr2, br2, wr3, br3, wt):
    nb = N // BN
    fsd = jax.ShapeDtypeStruct
    cspec = lambda: pl.BlockSpec((BN, CH), lambda i: (i, 0))
    cspec2 = lambda: pl.BlockSpec((BN, CH), lambda i: (i + N // BN, 0))
    full = lambda shape: pl.BlockSpec(shape, lambda i: tuple(0 for _ in shape))
    return pl.pallas_call(
        _k5_body,
        grid=(nb,),
        in_specs=[
            cspec(), cspec2(), cspec(), cspec2(),
            cspec(), cspec2(), cspec(), cspec2(),
            pl.BlockSpec((BN, 16), lambda i: (i, 0)),
            pl.BlockSpec((BN, 16), lambda i: (i + N // BN, 0)),
            pl.BlockSpec((BN, 9), lambda i: (i, 0)),
            pl.BlockSpec((BN, 3), lambda i: (i, 0)),
            pl.BlockSpec((BN, 1), lambda i: (i, 0)),
            full((HEADS * VC, H)), full((H, H)), full((H, H)),
            full((H, 2 * H)), full((2 * H,)), full((2 * H, H)), full((H,)),
            full((H, 6)), full((6,)), full((H, 1)),
        ],
        out_specs=[
            pl.BlockSpec((BN, 9), lambda i: (i, 0)),
            pl.BlockSpec((BN, 3), lambda i: (i, 0)),
            pl.BlockSpec((BN, SPH, H), lambda i: (i, 0, 0)),
        ],
        out_shape=[
            fsd((N, 9), jnp.float32),
            fsd((N, 3), jnp.float32),
            fsd((N, SPH, H), jnp.float32),
        ],
    )(a0, a0, a1, a1, a2, a2, a3, a3, den, den, rots9, trans, nm,
      wo, wffn1, wffn2, wr1, br1, wr2, br2, wr3, br3, wt)


def kernel(rots, trans, node_features, batch, x_mask, noising_mask,
           sampled_edge_index, seq_local_edge_index, W_edge, W_alpha, a_vec,
           W_v, W_o, W_ffn1, W_ffn2, W_rot1, b_rot1, W_rot2, b_rot2,
           W_rot3, b_rot3, W_t):
    del batch
    ei = jnp.concatenate([sampled_edge_index, seq_local_edge_index], axis=-1)
    src = jnp.pad(ei[0], (0, EP - E)).reshape(NROWS, EB)
    dst = jnp.pad(ei[1], (0, EP - E)).reshape(NROWS, EB)
    rots9 = rots.reshape(N, 9)
    nm = noising_mask.astype(jnp.float32).reshape(N, 1)
    xm = x_mask.astype(jnp.float32).reshape(N, 1)
    wad = W_alpha[:D]
    was = W_alpha[D:2 * D]
    wae = W_alpha[2 * D:]
    z144 = jnp.zeros((N, CH), jnp.float32)
    z16 = jnp.zeros((N, 16), jnp.float32)

    vt0, vt1, vt2, vt3, pd, ps, t16 = _k1(
        node_features, rots9, trans, nm, xm, wad, was, W_v)
    psum, vecd, dvec = _k2(pd, ps, t16, src, dst)
    (w16,) = _k3(psum, vecd, dvec, W_edge, wae, a_vec)
    a0, a1, a2, a3, den = _k4(vt0, vt1, vt2, vt3, w16, src, dst, z144, z16)
    rots9f, transf, bb = _k5(a0, a1, a2, a3, den, rots9, trans, nm,
                             W_o, W_ffn1, W_ffn2, W_rot1, b_rot1,
                             W_rot2, b_rot2, W_rot3, b_rot3, W_t)
    return rots9f.reshape(N, 3, 3), transf, bb


# speedup vs baseline: 7.6281x; 1.2707x over previous
import functools

import jax
import jax.numpy as jnp
from jax import lax
from jax.experimental import pallas as pl
from jax.experimental.pallas import tpu as pltpu
from jax.experimental.pallas import tpu_sc as plsc

N = 10000
H = 32
FC = 3
SPH = 9
HEADS = 8
ACH = 16
VC = 8
D = H + FC
E = 140000
EB = 64
NBLK = 70
NW = 32
EP = NW * NBLK * EB
NROWS = EP // EB
CH = 144
NCHUNK = 4
BN = 400
BE = 512

_IDEAL = ((-0.525, 1.363, 0.0), (0.0, 0.0, 0.0), (1.526, 0.0, 0.0))
import numpy as _np
_MU = _np.linspace(0.0, 20.0, 16).astype(_np.float32)
_FREQ = _np.exp(_np.arange(0, 16, 2).astype(_np.float32)
                * (-_np.log(10000.0) / 16.0)).astype(_np.float32)


def _k1_body(nf_r, rots9_r, trans_r, nm_r, xm_r, wad, was, wv,
             vt0, vt1, vt2, vt3, pd, ps, t16):
    nf = nf_r[...]
    rots9 = rots9_r[...]
    trans = trans_r[...]
    b = nf.shape[0]
    ed = nm_r[...] * (1.0 - xm_r[...])
    zc = jnp.zeros((b, 1), jnp.float32)
    row0 = jnp.concatenate([zc, zc, ed], axis=1)[:, None, :]
    rows = [row0]
    for i in range(3):
        r0 = rots9[:, 3 * i + 0]
        r1 = rots9[:, 3 * i + 1]
        ti = trans[:, i]
        fa0 = _IDEAL[0][0] * r0 + _IDEAL[0][1] * r1 + ti
        fa1 = ti
        fa2 = _IDEAL[2][0] * r0 + ti
        rows.append(jnp.stack([fa0, fa1, fa2], axis=1)[:, None, :])
    rows.append(jnp.zeros((b, 5, 3), jnp.float32))
    extra = jnp.concatenate(rows, axis=1)
    fused = jnp.concatenate([nf, extra], axis=2)
    x0 = fused[:, 0, :]
    pd[...] = jnp.dot(x0, wad[...], preferred_element_type=jnp.float32)
    ps[...] = jnp.dot(x0, was[...], preferred_element_type=jnp.float32)
    v = jnp.dot(fused.reshape(b * SPH, D), wv[...],
                preferred_element_type=jnp.float32).reshape(b, SPH, HEADS * VC)
    vt0[...] = jnp.concatenate([v[:, 0, :], v[:, 1, :], v[:, 2, :16]], axis=1)
    vt1[...] = jnp.concatenate([v[:, 2, 16:], v[:, 3, :], v[:, 4, :32]], axis=1)
    vt2[...] = jnp.concatenate([v[:, 4, 32:], v[:, 5, :], v[:, 6, :48]], axis=1)
    vt3[...] = jnp.concatenate([v[:, 6, 48:], v[:, 7, :], v[:, 8, :]], axis=1)
    t16[...] = jnp.concatenate([trans, jnp.zeros((b, 13), jnp.float32)], axis=1)


def _k1(nf, rots9, trans, nm, xm, wad, was, wv):
    nb = N // BN
    fsd = jax.ShapeDtypeStruct
    return pl.pallas_call(
        _k1_body,
        grid=(nb,),
        in_specs=[
            pl.BlockSpec((BN, SPH, H), lambda i: (i, 0, 0)),
            pl.BlockSpec((BN, 9), lambda i: (i, 0)),
            pl.BlockSpec((BN, 3), lambda i: (i, 0)),
            pl.BlockSpec((BN, 1), lambda i: (i, 0)),
            pl.BlockSpec((BN, 1), lambda i: (i, 0)),
            pl.BlockSpec((D, 128), lambda i: (0, 0)),
            pl.BlockSpec((D, 128), lambda i: (0, 0)),
            pl.BlockSpec((D, HEADS * VC), lambda i: (0, 0)),
        ],
        out_specs=[
            pl.BlockSpec((BN, CH), lambda i: (i, 0)),
            pl.BlockSpec((BN, CH), lambda i: (i, 0)),
            pl.BlockSpec((BN, CH), lambda i: (i, 0)),
            pl.BlockSpec((BN, CH), lambda i: (i, 0)),
            pl.BlockSpec((BN, 128), lambda i: (i, 0)),
            pl.BlockSpec((BN, 128), lambda i: (i, 0)),
            pl.BlockSpec((BN, 16), lambda i: (i, 0)),
        ],
        out_shape=[
            fsd((N, CH), jnp.float32), fsd((N, CH), jnp.float32),
            fsd((N, CH), jnp.float32), fsd((N, CH), jnp.float32),
            fsd((N, 128), jnp.float32), fsd((N, 128), jnp.float32),
            fsd((N, 16), jnp.float32),
        ],
    )(nf, rots9, trans, nm, xm, wad, was, wv)


def _k2_body(pd_hbm, ps_hbm, t16_hbm, src_hbm, dst_hbm,
             pdg_hbm, psg_hbm, tsg_hbm, tdg_hbm, dvec_hbm,
             idx_s, idx_d, a, bb, ts, td, dfbuf, sem, semo):
    c = lax.axis_index("c")
    s = lax.axis_index("s")
    wid = c * 16 + s

    def block(j, carry):
        row = wid * NBLK + j
        pltpu.sync_copy(src_hbm.at[row], idx_s)
        pltpu.sync_copy(dst_hbm.at[row], idx_d)
        c1 = pltpu.async_copy(pd_hbm.at[idx_d], a, sem)
        c2 = pltpu.async_copy(ps_hbm.at[idx_s], bb, sem)
        c3 = pltpu.async_copy(t16_hbm.at[idx_s], ts, sem)
        c4 = pltpu.async_copy(t16_hbm.at[idx_d], td, sem)
        for t in range(EB // 16):
            sv = idx_s[pl.ds(t * 16, 16)]
            dv = idx_d[pl.ds(t * 16, 16)]
            dfbuf[pl.ds(t * 16, 16)] = (sv - dv).astype(jnp.float32)
        c1.wait()
        c2.wait()
        c3.wait()
        c4.wait()
        sl = pl.ds(row * EB, EB)
        o1 = pltpu.async_copy(a, pdg_hbm.at[sl], semo)
        o2 = pltpu.async_copy(bb, psg_hbm.at[sl], semo)
        o3 = pltpu.async_copy(ts, tsg_hbm.at[sl], semo)
        o4 = pltpu.async_copy(td, tdg_hbm.at[sl], semo)
        o5 = pltpu.async_copy(dfbuf, dvec_hbm.at[sl], semo)
        o1.wait()
        o2.wait()
        o3.wait()
        o4.wait()
        o5.wait()
        return carry

    lax.fori_loop(0, NBLK, block, 0)


def _k2(pd, ps, t16, src2d, dst2d):
    fsd = jax.ShapeDtypeStruct
    kern = functools.partial(
        pl.kernel,
        out_type=[fsd((EP, 128), jnp.float32), fsd((EP, 128), jnp.float32),
                  fsd((EP, 16), jnp.float32), fsd((EP, 16), jnp.float32),
                  fsd((EP,), jnp.float32)],
        mesh=plsc.VectorSubcoreMesh(core_axis_name="c", subcore_axis_name="s"),
        compiler_params=pltpu.CompilerParams(use_tc_tiling_on_sc=False),
        scratch_types=[
            pltpu.VMEM((EB,), jnp.int32),
            pltpu.VMEM((EB,), jnp.int32),
            pltpu.VMEM((EB, 128), jnp.float32),
            pltpu.VMEM((EB, 128), jnp.float32),
            pltpu.VMEM((EB, 16), jnp.float32),
            pltpu.VMEM((EB, 16), jnp.float32),
            pltpu.VMEM((EB,), jnp.float32),
            pltpu.SemaphoreType.DMA,
            pltpu.SemaphoreType.DMA,
        ],
    )(_k2_body)
    return kern(pd, ps, t16, src2d, dst2d)


def _k3_body(pdg_r, psg_r, tsg_r, tdg_r, dfl_r, wedge, wae, avec,
             mu_r, fr_r, w16):
    psum = pdg_r[...] + psg_r[...]
    b = psum.shape[0]
    vecd = tsg_r[...] - tdg_r[...]
    vec = vecd[:, 0:3]
    dist = jnp.sqrt(jnp.sum(vec * vec, axis=1))
    dfloat = dfl_r[...]
    mu = mu_r[...][0]
    sigma = 20.0 / 16.0
    rbf = jnp.exp(-(((dist[:, None] - mu[None, :]) / sigma) ** 2))
    freq = fr_r[...][0]
    ang = dfloat[:, None] * freq[None, :]
    pe = jnp.concatenate([jnp.cos(ang), jnp.sin(ang)], axis=1)
    ef = jnp.concatenate([rbf, pe], axis=1)
    ee = jax.nn.relu(jnp.dot(ef, wedge[...],
                             preferred_element_type=jnp.float32))
    h = psum[...] + jnp.dot(ee, wae[...],
                            preferred_element_type=jnp.float32)
    h = jax.nn.leaky_relu(h)
    ha = h * avec[...]
    alpha = jnp.concatenate(
        [jnp.sum(ha[:, ACH * hd:ACH * (hd + 1)], axis=1, keepdims=True)
         for hd in range(HEADS)], axis=1)
    valid = jnp.isfinite(dist) & (dist > 1e-3)
    w = jnp.where(valid[:, None], jnp.exp(alpha), 0.0)
    w16[...] = jnp.concatenate([w, jnp.zeros((b, 8), jnp.float32)], axis=1)


def _k3(pdg, psg, tsg, tdg, dvec, wedge, wae, avec):
    nb = EP // BE
    mu = jnp.asarray(_MU, jnp.float32).reshape(1, 16)
    fr = jnp.asarray(_FREQ, jnp.float32).reshape(1, 8)
    return pl.pallas_call(
        _k3_body,
        grid=(nb,),
        in_specs=[
            pl.BlockSpec((BE, 128), lambda i: (i, 0)),
            pl.BlockSpec((BE, 128), lambda i: (i, 0)),
            pl.BlockSpec((BE, 16), lambda i: (i, 0)),
            pl.BlockSpec((BE, 16), lambda i: (i, 0)),
            pl.BlockSpec((BE,), lambda i: (i,)),
            pl.BlockSpec((H, H), lambda i: (0, 0)),
            pl.BlockSpec((H, 128), lambda i: (0, 0)),
            pl.BlockSpec((1, HEADS * ACH), lambda i: (0, 0)),
            pl.BlockSpec((1, 16), lambda i: (0, 0)),
            pl.BlockSpec((1, 8), lambda i: (0, 0)),
        ],
        out_specs=[pl.BlockSpec((BE, 16), lambda i: (i, 0))],
        out_shape=[jax.ShapeDtypeStruct((EP, 16), jnp.float32)],
    )(pdg, psg, tsg, tdg, dvec, wedge, wae,
      avec.reshape(1, HEADS * ACH), mu, fr)


def _k4_body(vt0, vt1, vt2, vt3, w16_hbm, src_hbm, dst_hbm, z144, z16,
             a0, a1, a2, a3, den,
             idx_s0, idx_s1, idx_d, vrows0, vrows1, wrows, acc, accd,
             sem0, sem1):
    c = lax.axis_index("c")
    s = lax.axis_index("s")
    wid = c * 16 + s
    rows_per_tile = N // 16
    vts = [vt0, vt1, vt2, vt3]
    outs = [a0, a1, a2, a3]
    lanes = lax.iota(jnp.int32, 16)

    for chunk in range(NCHUNK):
        pltpu.sync_copy(z144.at[pl.ds(s * rows_per_tile, rows_per_tile)],
                        acc.at[pl.ds(s * rows_per_tile, rows_per_tile)])
        if chunk == 0:
            pltpu.sync_copy(z16.at[pl.ds(s * rows_per_tile, rows_per_tile)],
                            accd.at[pl.ds(s * rows_per_tile, rows_per_tile)])
        plsc.subcore_barrier()

        def process(row, vr):
            pltpu.sync_copy(dst_hbm.at[row], idx_d)
            pltpu.sync_copy(w16_hbm.at[pl.ds(row * EB, EB)], wrows)

            def edge(i, carry2):
                wr = wrows[i, :]
                wvs = [jnp.where(lanes < 8, wr[2 * p], wr[2 * p + 1])
                       for p in range(4)]
                for jj in range(CH // 16):
                    p = (9 * chunk + jj) % 4
                    sl = pl.ds(jj * 16, 16)
                    vr[i, sl] = vr[i, sl] * wvs[p]
                return carry2

            lax.fori_loop(0, EB, edge, 0)
            pltpu.sync_copy(vr, acc.at[idx_d], add=True)
            if chunk == 0:
                pltpu.sync_copy(wrows, accd.at[idx_d], add=True)

        row0 = wid * NBLK
        pltpu.sync_copy(src_hbm.at[row0], idx_s0)
        pltpu.async_copy(vts[chunk].at[idx_s0], vrows0, sem0)

        def pair(t, carry):
            rowa = wid * NBLK + 2 * t
            rowb = rowa + 1
            pltpu.sync_copy(src_hbm.at[rowb], idx_s1)
            pltpu.async_copy(vts[chunk].at[idx_s1], vrows1, sem1)
            pltpu.make_async_copy(vts[chunk].at[idx_s0], vrows0, sem0).wait()
            process(rowa, vrows0)

            @pl.when(2 * t + 2 < NBLK)
            def _():
                pltpu.sync_copy(src_hbm.at[rowa + 2], idx_s0)
                pltpu.async_copy(vts[chunk].at[idx_s0], vrows0, sem0)

            pltpu.make_async_copy(vts[chunk].at[idx_s1], vrows1, sem1).wait()
            process(rowb, vrows1)
            return carry

        lax.fori_loop(0, NBLK // 2, pair, 0)
        plsc.subcore_barrier()
        pltpu.sync_copy(
            acc.at[pl.ds(s * rows_per_tile, rows_per_tile)],
            outs[chunk].at[pl.ds(c * N + s * rows_per_tile, rows_per_tile)])
        if chunk == 0:
            pltpu.sync_copy(
                accd.at[pl.ds(s * rows_per_tile, rows_per_tile)],
                den.at[pl.ds(c * N + s * rows_per_tile, rows_per_tile)])
        plsc.subcore_barrier()


def _k4(vt0, vt1, vt2, vt3, w16, src2d, dst2d, z144, z16):
    fsd = jax.ShapeDtypeStruct
    kern = functools.partial(
        pl.kernel,
        out_type=[
            fsd((2 * N, CH), jnp.float32), fsd((2 * N, CH), jnp.float32),
            fsd((2 * N, CH), jnp.float32), fsd((2 * N, CH), jnp.float32),
            fsd((2 * N, 16), jnp.float32),
        ],
        mesh=plsc.VectorSubcoreMesh(core_axis_name="c", subcore_axis_name="s"),
        compiler_params=pltpu.CompilerParams(use_tc_tiling_on_sc=False),
        scratch_types=[
            pltpu.VMEM((EB,), jnp.int32),
            pltpu.VMEM((EB,), jnp.int32),
            pltpu.VMEM((EB,), jnp.int32),
            pltpu.VMEM((EB, CH), jnp.float32),
            pltpu.VMEM((EB, CH), jnp.float32),
            pltpu.VMEM((EB, 16), jnp.float32),
            pltpu.VMEM_SHARED((N, CH), jnp.float32),
            pltpu.VMEM_SHARED((N, 16), jnp.float32),
            pltpu.SemaphoreType.DMA,
            pltpu.SemaphoreType.DMA,
        ],
    )(_k4_body)
    return kern(vt0, vt1, vt2, vt3, w16, src2d, dst2d, z144, z16)


def _k5_body(a0a, a0b, a1a, a1b, a2a, a2b, a3a, a3b, dena, denb,
             rots9, trans, nm, wo, wffn1, wffn2,
             wr1, br1, wr2, br2, wr3, br3, wt,
             rots_out, trans_out, bb_out):
    b = rots9.shape[0]
    den = dena[...] + denb[...]
    den8 = den[:, :HEADS] + 1e-9
    chunks = []
    for (pa, pb, k) in ((a0a, a0b, 0), (a1a, a1b, 1),
                        (a2a, a2b, 2), (a3a, a3b, 3)):
        ck = pa[...] + pb[...]
        dpieces = [jnp.broadcast_to(den8[:, (18 * k + t) % 8][:, None],
                                    (b, 8)) for t in range(18)]
        ck = ck / jnp.concatenate(dpieces, axis=1)
        chunks.append(ck)
    c0, c1, c2, c3 = chunks
    planes = [
        c0[:, 0:64], c0[:, 64:128],
        jnp.concatenate([c0[:, 128:144], c1[:, 0:48]], axis=1),
        c1[:, 48:112],
        jnp.concatenate([c1[:, 112:144], c2[:, 0:32]], axis=1),
        c2[:, 32:96],
        jnp.concatenate([c2[:, 96:144], c3[:, 0:16]], axis=1),
        c3[:, 16:80], c3[:, 80:144],
    ]
    agg = jnp.concatenate([p[:, None, :] for p in planes], axis=1)
    out = jnp.dot(agg.reshape(b * SPH, HEADS * VC), wo[...],
                  preferred_element_type=jnp.float32).reshape(b, SPH, H)
    ff = jnp.dot(out.reshape(b * SPH, H), wffn1[...],
                 preferred_element_type=jnp.float32).reshape(b, SPH, H)
    ff0 = jax.nn.gelu(ff[:, 0, :])
    ff = jnp.concatenate([ff0[:, None, :], ff[:, 1:, :]], axis=1)
    bb = out + jnp.dot(ff.reshape(b * SPH, H), wffn2[...],
                       preferred_element_type=jnp.float32).reshape(b, SPH, H)
    bb_out[...] = bb
    inv = bb[:, 0, :]
    h1 = jax.nn.relu(jnp.dot(inv, wr1[...],
                             preferred_element_type=jnp.float32) + br1[...])
    h2 = jax.nn.relu(jnp.dot(h1, wr2[...],
                             preferred_element_type=jnp.float32) + br2[...])
    qv = jnp.dot(h2, wr3[...], preferred_element_type=jnp.float32) + br3[...]
    qx, qy, qz = qv[:, 0], qv[:, 1], qv[:, 2]
    qn = jnp.sqrt(1.0 + qx * qx + qy * qy + qz * qz)
    w_, x_, y_, z_ = 1.0 / qn, qx / qn, qy / qn, qz / qn
    u = [1 - 2 * (y_ * y_ + z_ * z_), 2 * (x_ * y_ - w_ * z_), 2 * (x_ * z_ + w_ * y_),
         2 * (x_ * y_ + w_ * z_), 1 - 2 * (x_ * x_ + z_ * z_), 2 * (y_ * z_ - w_ * x_),
         2 * (x_ * z_ - w_ * y_), 2 * (y_ * z_ + w_ * x_), 1 - 2 * (x_ * x_ + y_ * y_)]
    tupd = jnp.sum(bb[:, 1:4, :] * wt[...].reshape(1, 1, H), axis=2)
    r = rots9[...]
    del rots9
    ru = []
    for i in range(3):
        for kk in range(3):
            ru.append(r[:, 3 * i + 0] * u[0 + kk] + r[:, 3 * i + 1] * u[3 + kk]
                      + r[:, 3 * i + 2] * u[6 + kk])
    rotsu = jnp.stack(ru, axis=1)
    mk = nm[...]
    rots_out[...] = mk * rotsu + (1.0 - mk) * r
    trans_out[...] = trans[...] + mk * tupd


def _k5(a0, a1, a2, a3, den, rots9, trans, nm, wo, wffn1, wffn2,
        wr1, br1, wr2, br2, wr3, br3, wt):
    nb = N // BN
    fsd = jax.ShapeDtypeStruct
    cspec = lambda: pl.BlockSpec((BN, CH), lambda i: (i, 0))
    cspec2 = lambda: pl.BlockSpec((BN, CH), lambda i: (i + N // BN, 0))
    full = lambda shape: pl.BlockSpec(shape, lambda i: tuple(0 for _ in shape))
    return pl.pallas_call(
        _k5_body,
        grid=(nb,),
        in_specs=[
            cspec(), cspec2(), cspec(), cspec2(),
            cspec(), cspec2(), cspec(), cspec2(),
            pl.BlockSpec((BN, 16), lambda i: (i, 0)),
            pl.BlockSpec((BN, 16), lambda i: (i + N // BN, 0)),
            pl.BlockSpec((BN, 9), lambda i: (i, 0)),
            pl.BlockSpec((BN, 3), lambda i: (i, 0)),
            pl.BlockSpec((BN, 1), lambda i: (i, 0)),
            full((HEADS * VC, H)), full((H, H)), full((H, H)),
            full((H, 2 * H)), full((2 * H,)), full((2 * H, H)), full((H,)),
            full((H, 6)), full((6,)), full((H, 1)),
        ],
        out_specs=[
            pl.BlockSpec((BN, 9), lambda i: (i, 0)),
            pl.BlockSpec((BN, 3), lambda i: (i, 0)),
            pl.BlockSpec((BN, SPH, H), lambda i: (i, 0, 0)),
        ],
        out_shape=[
            fsd((N, 9), jnp.float32),
            fsd((N, 3), jnp.float32),
            fsd((N, SPH, H), jnp.float32),
        ],
    )(a0, a0, a1, a1, a2, a2, a3, a3, den, den, rots9, trans, nm,
      wo, wffn1, wffn2, wr1, br1, wr2, br2, wr3, br3, wt)


def kernel(rots, trans, node_features, batch, x_mask, noising_mask,
           sampled_edge_index, seq_local_edge_index, W_edge, W_alpha, a_vec,
           W_v, W_o, W_ffn1, W_ffn2, W_rot1, b_rot1, W_rot2, b_rot2,
           W_rot3, b_rot3, W_t):
    del batch
    ei = jnp.concatenate([sampled_edge_index, seq_local_edge_index], axis=-1)
    src = jnp.pad(ei[0], (0, EP - E)).reshape(NROWS, EB)
    dst = jnp.pad(ei[1], (0, EP - E)).reshape(NROWS, EB)
    rots9 = rots.reshape(N, 9)
    nm = noising_mask.astype(jnp.float32).reshape(N, 1)
    xm = x_mask.astype(jnp.float32).reshape(N, 1)
    wad = W_alpha[:D]
    was = W_alpha[D:2 * D]
    wae = W_alpha[2 * D:]
    z144 = jnp.zeros((N, CH), jnp.float32)
    z16 = jnp.zeros((N, 16), jnp.float32)

    vt0, vt1, vt2, vt3, pd, ps, t16 = _k1(
        node_features, rots9, trans, nm, xm, wad, was, W_v)
    pdg, psg, tsg, tdg, dvec = _k2(pd, ps, t16, src, dst)
    (w16,) = _k3(pdg, psg, tsg, tdg, dvec, W_edge, wae, a_vec)
    a0, a1, a2, a3, den = _k4(vt0, vt1, vt2, vt3, w16, src, dst, z144, z16)
    rots9f, transf, bb = _k5(a0, a1, a2, a3, den, rots9, trans, nm,
                             W_o, W_ffn1, W_ffn2, W_rot1, b_rot1,
                             W_rot2, b_rot2, W_rot3, b_rot3, W_t)
    return rots9f.reshape(N, 3, 3), transf, bb


# speedup vs baseline: 9.8473x; 1.2909x over previous
import functools

import jax
import jax.numpy as jnp
from jax import lax
from jax.experimental import pallas as pl
from jax.experimental.pallas import tpu as pltpu
from jax.experimental.pallas import tpu_sc as plsc

N = 10000
H = 32
FC = 3
SPH = 9
HEADS = 8
ACH = 16
VC = 8
D = H + FC
E = 140000
EB = 64
NBLK = 70
NW = 32
EP = NW * NBLK * EB
NROWS = EP // EB
CH = 144
NCHUNK = 4
BN = 400
BE = 512

_IDEAL = ((-0.525, 1.363, 0.0), (0.0, 0.0, 0.0), (1.526, 0.0, 0.0))
import numpy as _np
_MU = _np.linspace(0.0, 20.0, 16).astype(_np.float32)
_FREQ = _np.exp(_np.arange(0, 16, 2).astype(_np.float32)
                * (-_np.log(10000.0) / 16.0)).astype(_np.float32)


def _k1_body(nf_r, rots9_r, trans_r, nm_r, xm_r, wad, was, wv,
             vt0, vt1, vt2, vt3, pd, ps, t16):
    nf = nf_r[...]
    rots9 = rots9_r[...]
    trans = trans_r[...]
    b = nf.shape[0]
    ed = nm_r[...] * (1.0 - xm_r[...])
    zc = jnp.zeros((b, 1), jnp.float32)
    row0 = jnp.concatenate([zc, zc, ed], axis=1)[:, None, :]
    rows = [row0]
    for i in range(3):
        r0 = rots9[:, 3 * i + 0]
        r1 = rots9[:, 3 * i + 1]
        ti = trans[:, i]
        fa0 = _IDEAL[0][0] * r0 + _IDEAL[0][1] * r1 + ti
        fa1 = ti
        fa2 = _IDEAL[2][0] * r0 + ti
        rows.append(jnp.stack([fa0, fa1, fa2], axis=1)[:, None, :])
    rows.append(jnp.zeros((b, 5, 3), jnp.float32))
    extra = jnp.concatenate(rows, axis=1)
    fused = jnp.concatenate([nf, extra], axis=2)
    x0 = fused[:, 0, :]
    pd[...] = jnp.dot(x0, wad[...], preferred_element_type=jnp.float32)
    ps[...] = jnp.dot(x0, was[...], preferred_element_type=jnp.float32)
    v = jnp.dot(fused.reshape(b * SPH, D), wv[...],
                preferred_element_type=jnp.float32).reshape(b, SPH, HEADS * VC)
    vt0[...] = jnp.concatenate([v[:, 0, :], v[:, 1, :], v[:, 2, :16]], axis=1)
    vt1[...] = jnp.concatenate([v[:, 2, 16:], v[:, 3, :], v[:, 4, :32]], axis=1)
    vt2[...] = jnp.concatenate([v[:, 4, 32:], v[:, 5, :], v[:, 6, :48]], axis=1)
    vt3[...] = jnp.concatenate([v[:, 6, 48:], v[:, 7, :], v[:, 8, :]], axis=1)
    t16[...] = jnp.concatenate([trans, jnp.zeros((b, 13), jnp.float32)], axis=1)


def _k1(nf, rots9, trans, nm, xm, wad, was, wv):
    nb = N // BN
    fsd = jax.ShapeDtypeStruct
    return pl.pallas_call(
        _k1_body,
        grid=(nb,),
        in_specs=[
            pl.BlockSpec((BN, SPH, H), lambda i: (i, 0, 0)),
            pl.BlockSpec((BN, 9), lambda i: (i, 0)),
            pl.BlockSpec((BN, 3), lambda i: (i, 0)),
            pl.BlockSpec((BN, 1), lambda i: (i, 0)),
            pl.BlockSpec((BN, 1), lambda i: (i, 0)),
            pl.BlockSpec((D, 128), lambda i: (0, 0)),
            pl.BlockSpec((D, 128), lambda i: (0, 0)),
            pl.BlockSpec((D, HEADS * VC), lambda i: (0, 0)),
        ],
        out_specs=[
            pl.BlockSpec((BN, CH), lambda i: (i, 0)),
            pl.BlockSpec((BN, CH), lambda i: (i, 0)),
            pl.BlockSpec((BN, CH), lambda i: (i, 0)),
            pl.BlockSpec((BN, CH), lambda i: (i, 0)),
            pl.BlockSpec((BN, 128), lambda i: (i, 0)),
            pl.BlockSpec((BN, 128), lambda i: (i, 0)),
            pl.BlockSpec((BN, 16), lambda i: (i, 0)),
        ],
        out_shape=[
            fsd((N, CH), jnp.float32), fsd((N, CH), jnp.float32),
            fsd((N, CH), jnp.float32), fsd((N, CH), jnp.float32),
            fsd((N, 128), jnp.float32), fsd((N, 128), jnp.float32),
            fsd((N, 16), jnp.float32),
        ],
    )(nf, rots9, trans, nm, xm, wad, was, wv)


def _k2_body(pd_hbm, ps_hbm, t16_hbm, src_hbm, dst_hbm,
             pdg_hbm, psg_hbm, tsg_hbm, tdg_hbm, dvec_hbm,
             idx_s, idx_d, a, bb, ts, td, dfbuf, sem, semo):
    c = lax.axis_index("c")
    s = lax.axis_index("s")
    wid = c * 16 + s

    def block(j, carry):
        row = wid * NBLK + j
        pltpu.sync_copy(src_hbm.at[row], idx_s)
        pltpu.sync_copy(dst_hbm.at[row], idx_d)
        c1 = pltpu.async_copy(pd_hbm.at[idx_d], a, sem)
        c2 = pltpu.async_copy(ps_hbm.at[idx_s], bb, sem)
        c3 = pltpu.async_copy(t16_hbm.at[idx_s], ts, sem)
        c4 = pltpu.async_copy(t16_hbm.at[idx_d], td, sem)
        for t in range(EB // 16):
            sv = idx_s[pl.ds(t * 16, 16)]
            dv = idx_d[pl.ds(t * 16, 16)]
            dfbuf[pl.ds(t * 16, 16)] = (sv - dv).astype(jnp.float32)
        c1.wait()
        c2.wait()
        c3.wait()
        c4.wait()
        sl = pl.ds(row * EB, EB)
        o1 = pltpu.async_copy(a, pdg_hbm.at[sl], semo)
        o2 = pltpu.async_copy(bb, psg_hbm.at[sl], semo)
        o3 = pltpu.async_copy(ts, tsg_hbm.at[sl], semo)
        o4 = pltpu.async_copy(td, tdg_hbm.at[sl], semo)
        o5 = pltpu.async_copy(dfbuf, dvec_hbm.at[sl], semo)
        o1.wait()
        o2.wait()
        o3.wait()
        o4.wait()
        o5.wait()
        return carry

    lax.fori_loop(0, NBLK, block, 0)


def _k2(pd, ps, t16, src2d, dst2d):
    fsd = jax.ShapeDtypeStruct
    kern = functools.partial(
        pl.kernel,
        out_type=[fsd((EP, 128), jnp.float32), fsd((EP, 128), jnp.float32),
                  fsd((EP, 16), jnp.float32), fsd((EP, 16), jnp.float32),
                  fsd((EP,), jnp.float32)],
        mesh=plsc.VectorSubcoreMesh(core_axis_name="c", subcore_axis_name="s"),
        compiler_params=pltpu.CompilerParams(use_tc_tiling_on_sc=False),
        scratch_types=[
            pltpu.VMEM((EB,), jnp.int32),
            pltpu.VMEM((EB,), jnp.int32),
            pltpu.VMEM((EB, 128), jnp.float32),
            pltpu.VMEM((EB, 128), jnp.float32),
            pltpu.VMEM((EB, 16), jnp.float32),
            pltpu.VMEM((EB, 16), jnp.float32),
            pltpu.VMEM((EB,), jnp.float32),
            pltpu.SemaphoreType.DMA,
            pltpu.SemaphoreType.DMA,
        ],
    )(_k2_body)
    return kern(pd, ps, t16, src2d, dst2d)


def _k3_body(pdg_r, psg_r, tsg_r, tdg_r, dfl_r, wedge, wae, avec,
             mu_r, fr_r, w16):
    psum = pdg_r[...] + psg_r[...]
    b = psum.shape[0]
    vecd = tsg_r[...] - tdg_r[...]
    vec = vecd[:, 0:3]
    dist = jnp.sqrt(jnp.sum(vec * vec, axis=1))
    dfloat = dfl_r[...]
    mu = mu_r[...][0]
    sigma = 20.0 / 16.0
    rbf = jnp.exp(-(((dist[:, None] - mu[None, :]) / sigma) ** 2))
    freq = fr_r[...][0]
    ang = dfloat[:, None] * freq[None, :]
    pe = jnp.concatenate([jnp.cos(ang), jnp.sin(ang)], axis=1)
    ef = jnp.concatenate([rbf, pe], axis=1)
    ee = jax.nn.relu(jnp.dot(ef, wedge[...],
                             preferred_element_type=jnp.float32))
    h = psum[...] + jnp.dot(ee, wae[...],
                            preferred_element_type=jnp.float32)
    h = jax.nn.leaky_relu(h)
    alpha = jnp.dot(h, avec[...],
                    preferred_element_type=jnp.float32)[:, :HEADS]
    valid = jnp.isfinite(dist) & (dist > 1e-3)
    w = jnp.where(valid[:, None], jnp.exp(alpha), 0.0)
    w16[...] = jnp.concatenate([w, jnp.zeros((b, 8), jnp.float32)], axis=1)


def _k3(pdg, psg, tsg, tdg, dvec, wedge, wae, avec):
    nb = EP // BE
    mu = jnp.asarray(_MU, jnp.float32).reshape(1, 16)
    fr = jnp.asarray(_FREQ, jnp.float32).reshape(1, 8)
    return pl.pallas_call(
        _k3_body,
        grid=(nb,),
        in_specs=[
            pl.BlockSpec((BE, 128), lambda i: (i, 0)),
            pl.BlockSpec((BE, 128), lambda i: (i, 0)),
            pl.BlockSpec((BE, 16), lambda i: (i, 0)),
            pl.BlockSpec((BE, 16), lambda i: (i, 0)),
            pl.BlockSpec((BE,), lambda i: (i,)),
            pl.BlockSpec((H, H), lambda i: (0, 0)),
            pl.BlockSpec((H, 128), lambda i: (0, 0)),
            pl.BlockSpec((HEADS * ACH, 16), lambda i: (0, 0)),
            pl.BlockSpec((1, 16), lambda i: (0, 0)),
            pl.BlockSpec((1, 8), lambda i: (0, 0)),
        ],
        out_specs=[pl.BlockSpec((BE, 16), lambda i: (i, 0))],
        out_shape=[jax.ShapeDtypeStruct((EP, 16), jnp.float32)],
    )(pdg, psg, tsg, tdg, dvec, wedge, wae, avec, mu, fr)


def _k4_body(vt0, vt1, vt2, vt3, w16_hbm, src_hbm, dst_hbm, z144, z16,
             a0, a1, a2, a3, den,
             idx_s0, idx_s1, idx_d, vrows0, vrows1, wrows, acc, accd,
             sem0, sem1):
    c = lax.axis_index("c")
    s = lax.axis_index("s")
    wid = c * 16 + s
    rows_per_tile = N // 16
    vts = [vt0, vt1, vt2, vt3]
    outs = [a0, a1, a2, a3]
    lanes = lax.iota(jnp.int32, 16)

    for chunk in range(NCHUNK):
        pltpu.sync_copy(z144.at[pl.ds(s * rows_per_tile, rows_per_tile)],
                        acc.at[pl.ds(s * rows_per_tile, rows_per_tile)])
        if chunk == 0:
            pltpu.sync_copy(z16.at[pl.ds(s * rows_per_tile, rows_per_tile)],
                            accd.at[pl.ds(s * rows_per_tile, rows_per_tile)])
        plsc.subcore_barrier()

        def process(row, vr):
            pltpu.sync_copy(dst_hbm.at[row], idx_d)
            pltpu.sync_copy(w16_hbm.at[pl.ds(row * EB, EB)], wrows)

            def edge(i, carry2):
                wr = wrows[i, :]
                wvs = [jnp.where(lanes < 8, wr[2 * p], wr[2 * p + 1])
                       for p in range(4)]
                for jj in range(CH // 16):
                    p = (9 * chunk + jj) % 4
                    sl = pl.ds(jj * 16, 16)
                    vr[i, sl] = vr[i, sl] * wvs[p]
                return carry2

            lax.fori_loop(0, EB, edge, 0)
            pltpu.sync_copy(vr, acc.at[idx_d], add=True)
            if chunk == 0:
                pltpu.sync_copy(wrows, accd.at[idx_d], add=True)

        row0 = wid * NBLK
        pltpu.sync_copy(src_hbm.at[row0], idx_s0)
        pltpu.async_copy(vts[chunk].at[idx_s0], vrows0, sem0)

        def pair(t, carry):
            rowa = wid * NBLK + 2 * t
            rowb = rowa + 1
            pltpu.sync_copy(src_hbm.at[rowb], idx_s1)
            pltpu.async_copy(vts[chunk].at[idx_s1], vrows1, sem1)
            pltpu.make_async_copy(vts[chunk].at[idx_s0], vrows0, sem0).wait()
            process(rowa, vrows0)

            @pl.when(2 * t + 2 < NBLK)
            def _():
                pltpu.sync_copy(src_hbm.at[rowa + 2], idx_s0)
                pltpu.async_copy(vts[chunk].at[idx_s0], vrows0, sem0)

            pltpu.make_async_copy(vts[chunk].at[idx_s1], vrows1, sem1).wait()
            process(rowb, vrows1)
            return carry

        lax.fori_loop(0, NBLK // 2, pair, 0)
        plsc.subcore_barrier()
        pltpu.sync_copy(
            acc.at[pl.ds(s * rows_per_tile, rows_per_tile)],
            outs[chunk].at[pl.ds(c * N + s * rows_per_tile, rows_per_tile)])
        if chunk == 0:
            pltpu.sync_copy(
                accd.at[pl.ds(s * rows_per_tile, rows_per_tile)],
                den.at[pl.ds(c * N + s * rows_per_tile, rows_per_tile)])
        plsc.subcore_barrier()


def _k4(vt0, vt1, vt2, vt3, w16, src2d, dst2d, z144, z16):
    fsd = jax.ShapeDtypeStruct
    kern = functools.partial(
        pl.kernel,
        out_type=[
            fsd((2 * N, CH), jnp.float32), fsd((2 * N, CH), jnp.float32),
            fsd((2 * N, CH), jnp.float32), fsd((2 * N, CH), jnp.float32),
            fsd((2 * N, 16), jnp.float32),
        ],
        mesh=plsc.VectorSubcoreMesh(core_axis_name="c", subcore_axis_name="s"),
        compiler_params=pltpu.CompilerParams(use_tc_tiling_on_sc=False),
        scratch_types=[
            pltpu.VMEM((EB,), jnp.int32),
            pltpu.VMEM((EB,), jnp.int32),
            pltpu.VMEM((EB,), jnp.int32),
            pltpu.VMEM((EB, CH), jnp.float32),
            pltpu.VMEM((EB, CH), jnp.float32),
            pltpu.VMEM((EB, 16), jnp.float32),
            pltpu.VMEM_SHARED((N, CH), jnp.float32),
            pltpu.VMEM_SHARED((N, 16), jnp.float32),
            pltpu.SemaphoreType.DMA,
            pltpu.SemaphoreType.DMA,
        ],
    )(_k4_body)
    return kern(vt0, vt1, vt2, vt3, w16, src2d, dst2d, z144, z16)


def _k5_body(a0a, a0b, a1a, a1b, a2a, a2b, a3a, a3b, dena, denb,
             rots9, trans, nm, wo, wffn1, wffn2,
             wr1, br1, wr2, br2, wr3, br3, wt,
             rots_out, trans_out, bb_out):
    b = rots9.shape[0]
    den = dena[...] + denb[...]
    den8 = den[:, :HEADS] + 1e-9
    chunks = []
    for (pa, pb, k) in ((a0a, a0b, 0), (a1a, a1b, 1),
                        (a2a, a2b, 2), (a3a, a3b, 3)):
        ck = pa[...] + pb[...]
        dpieces = [jnp.broadcast_to(den8[:, (18 * k + t) % 8][:, None],
                                    (b, 8)) for t in range(18)]
        ck = ck / jnp.concatenate(dpieces, axis=1)
        chunks.append(ck)
    aggf = jnp.concatenate(chunks, axis=1)
    outf = jnp.dot(aggf, wo[...], preferred_element_type=jnp.float32)
    fff = jnp.dot(outf, wffn1[...], preferred_element_type=jnp.float32)
    ff0 = jax.nn.gelu(fff[:, :H])
    fff = jnp.concatenate([ff0, fff[:, H:]], axis=1)
    bbf = outf + jnp.dot(fff, wffn2[...], preferred_element_type=jnp.float32)
    bb_out[...] = bbf
    inv = bbf[:, :H]
    h1 = jax.nn.relu(jnp.dot(inv, wr1[...],
                             preferred_element_type=jnp.float32) + br1[...])
    h2 = jax.nn.relu(jnp.dot(h1, wr2[...],
                             preferred_element_type=jnp.float32) + br2[...])
    qv = jnp.dot(h2, wr3[...], preferred_element_type=jnp.float32) + br3[...]
    qx, qy, qz = qv[:, 0], qv[:, 1], qv[:, 2]
    qn = jnp.sqrt(1.0 + qx * qx + qy * qy + qz * qz)
    w_, x_, y_, z_ = 1.0 / qn, qx / qn, qy / qn, qz / qn
    u = [1 - 2 * (y_ * y_ + z_ * z_), 2 * (x_ * y_ - w_ * z_), 2 * (x_ * z_ + w_ * y_),
         2 * (x_ * y_ + w_ * z_), 1 - 2 * (x_ * x_ + z_ * z_), 2 * (y_ * z_ - w_ * x_),
         2 * (x_ * z_ - w_ * y_), 2 * (y_ * z_ + w_ * x_), 1 - 2 * (x_ * x_ + y_ * y_)]
    tupd = jnp.dot(bbf, wt[...], preferred_element_type=jnp.float32)
    r = rots9[...]
    del rots9
    ru = []
    for i in range(3):
        for kk in range(3):
            ru.append(r[:, 3 * i + 0] * u[0 + kk] + r[:, 3 * i + 1] * u[3 + kk]
                      + r[:, 3 * i + 2] * u[6 + kk])
    rotsu = jnp.stack(ru, axis=1)
    mk = nm[...]
    rots_out[...] = mk * rotsu + (1.0 - mk) * r
    trans_out[...] = trans[...] + mk * tupd


def _k5(a0, a1, a2, a3, den, rots9, trans, nm, wo, wffn1, wffn2,
        wr1, br1, wr2, br2, wr3, br3, wt):
    nb = N // BN
    fsd = jax.ShapeDtypeStruct
    cspec = lambda: pl.BlockSpec((BN, CH), lambda i: (i, 0))
    cspec2 = lambda: pl.BlockSpec((BN, CH), lambda i: (i + N // BN, 0))
    full = lambda shape: pl.BlockSpec(shape, lambda i: tuple(0 for _ in shape))
    return pl.pallas_call(
        _k5_body,
        grid=(nb,),
        in_specs=[
            cspec(), cspec2(), cspec(), cspec2(),
            cspec(), cspec2(), cspec(), cspec2(),
            pl.BlockSpec((BN, 16), lambda i: (i, 0)),
            pl.BlockSpec((BN, 16), lambda i: (i + N // BN, 0)),
            pl.BlockSpec((BN, 9), lambda i: (i, 0)),
            pl.BlockSpec((BN, 3), lambda i: (i, 0)),
            pl.BlockSpec((BN, 1), lambda i: (i, 0)),
            full((NCHUNK * CH, SPH * H)), full((SPH * H, SPH * H)),
            full((SPH * H, SPH * H)),
            full((H, 2 * H)), full((2 * H,)), full((2 * H, H)), full((H,)),
            full((H, 6)), full((6,)), full((SPH * H, 3)),
        ],
        out_specs=[
            pl.BlockSpec((BN, 9), lambda i: (i, 0)),
            pl.BlockSpec((BN, 3), lambda i: (i, 0)),
            pl.BlockSpec((BN, SPH * H), lambda i: (i, 0)),
        ],
        out_shape=[
            fsd((N, 9), jnp.float32),
            fsd((N, 3), jnp.float32),
            fsd((N, SPH * H), jnp.float32),
        ],
    )(a0, a0, a1, a1, a2, a2, a3, a3, den, den, rots9, trans, nm,
      wo, wffn1, wffn2, wr1, br1, wr2, br2, wr3, br3, wt)


def kernel(rots, trans, node_features, batch, x_mask, noising_mask,
           sampled_edge_index, seq_local_edge_index, W_edge, W_alpha, a_vec,
           W_v, W_o, W_ffn1, W_ffn2, W_rot1, b_rot1, W_rot2, b_rot2,
           W_rot3, b_rot3, W_t):
    del batch
    ei = jnp.concatenate([sampled_edge_index, seq_local_edge_index], axis=-1)
    src = jnp.pad(ei[0], (0, EP - E)).reshape(NROWS, EB)
    dst = jnp.pad(ei[1], (0, EP - E)).reshape(NROWS, EB)
    rots9 = rots.reshape(N, 9)
    nm = noising_mask.astype(jnp.float32).reshape(N, 1)
    xm = x_mask.astype(jnp.float32).reshape(N, 1)
    wad = W_alpha[:D]
    was = W_alpha[D:2 * D]
    wae = W_alpha[2 * D:]
    a16 = jnp.zeros((HEADS * ACH, 16), jnp.float32).at[
        jnp.arange(HEADS * ACH), jnp.arange(HEADS * ACH) // ACH
    ].set(a_vec.reshape(-1))
    eye9 = jnp.eye(SPH, dtype=jnp.float32)
    wo_bd = jnp.kron(eye9, W_o)
    wffn1_bd = jnp.kron(eye9, W_ffn1)
    wffn2_bd = jnp.kron(eye9, W_ffn2)
    p93 = jnp.zeros((SPH, 3), jnp.float32).at[jnp.arange(1, 4),
                                              jnp.arange(3)].set(1.0)
    wt_bd = jnp.kron(p93, W_t)
    z144 = jnp.zeros((N, CH), jnp.float32)
    z16 = jnp.zeros((N, 16), jnp.float32)

    vt0, vt1, vt2, vt3, pd, ps, t16 = _k1(
        node_features, rots9, trans, nm, xm, wad, was, W_v)
    pdg, psg, tsg, tdg, dvec = _k2(pd, ps, t16, src, dst)
    (w16,) = _k3(pdg, psg, tsg, tdg, dvec, W_edge, wae, a16)
    a0, a1, a2, a3, den = _k4(vt0, vt1, vt2, vt3, w16, src, dst, z144, z16)
    rots9f, transf, bb = _k5(a0, a1, a2, a3, den, rots9, trans, nm,
                             wo_bd, wffn1_bd, wffn2_bd, W_rot1, b_rot1,
                             W_rot2, b_rot2, W_rot3, b_rot3, wt_bd)
    return rots9f.reshape(N, 3, 3), transf, bb.reshape(N, SPH, H)


# speedup vs baseline: 10.6783x; 1.0844x over previous
import functools

import jax
import jax.numpy as jnp
from jax import lax
from jax.experimental import pallas as pl
from jax.experimental.pallas import tpu as pltpu
from jax.experimental.pallas import tpu_sc as plsc

N = 10000
H = 32
FC = 3
SPH = 9
HEADS = 8
ACH = 16
VC = 8
D = H + FC
E = 140000
EB = 64
NBLK = 70
NW = 32
EP = NW * NBLK * EB
NROWS = EP // EB
CH = 144
NCHUNK = 4
BN = 400
BE = 2048

_IDEAL = ((-0.525, 1.363, 0.0), (0.0, 0.0, 0.0), (1.526, 0.0, 0.0))
import numpy as _np
_MU = _np.linspace(0.0, 20.0, 16).astype(_np.float32)
_FREQ = _np.exp(_np.arange(0, 16, 2).astype(_np.float32)
                * (-_np.log(10000.0) / 16.0)).astype(_np.float32)


def _k1_body(nf_r, rots9_r, trans_r, nm_r, xm_r, wad, was, wv,
             vt0, vt1, vt2, vt3, pd, ps, t16):
    nf = nf_r[...]
    rots9 = rots9_r[...]
    trans = trans_r[...]
    b = nf.shape[0]
    ed = nm_r[...] * (1.0 - xm_r[...])
    zc = jnp.zeros((b, 1), jnp.float32)
    row0 = jnp.concatenate([zc, zc, ed], axis=1)[:, None, :]
    rows = [row0]
    for i in range(3):
        r0 = rots9[:, 3 * i + 0]
        r1 = rots9[:, 3 * i + 1]
        ti = trans[:, i]
        fa0 = _IDEAL[0][0] * r0 + _IDEAL[0][1] * r1 + ti
        fa1 = ti
        fa2 = _IDEAL[2][0] * r0 + ti
        rows.append(jnp.stack([fa0, fa1, fa2], axis=1)[:, None, :])
    rows.append(jnp.zeros((b, 5, 3), jnp.float32))
    extra = jnp.concatenate(rows, axis=1)
    fused = jnp.concatenate([nf, extra], axis=2)
    x0 = fused[:, 0, :]
    pd[...] = jnp.dot(x0, wad[...], preferred_element_type=jnp.float32)
    ps[...] = jnp.dot(x0, was[...], preferred_element_type=jnp.float32)
    v = jnp.dot(fused.reshape(b * SPH, D), wv[...],
                preferred_element_type=jnp.float32).reshape(b, SPH, HEADS * VC)
    vt0[...] = jnp.concatenate([v[:, 0, :], v[:, 1, :], v[:, 2, :16]], axis=1)
    vt1[...] = jnp.concatenate([v[:, 2, 16:], v[:, 3, :], v[:, 4, :32]], axis=1)
    vt2[...] = jnp.concatenate([v[:, 4, 32:], v[:, 5, :], v[:, 6, :48]], axis=1)
    vt3[...] = jnp.concatenate([v[:, 6, 48:], v[:, 7, :], v[:, 8, :]], axis=1)
    t16[...] = jnp.concatenate([trans, jnp.zeros((b, 13), jnp.float32)], axis=1)


def _k1(nf, rots9, trans, nm, xm, wad, was, wv):
    nb = N // BN
    fsd = jax.ShapeDtypeStruct
    return pl.pallas_call(
        _k1_body,
        grid=(nb,),
        in_specs=[
            pl.BlockSpec((BN, SPH, H), lambda i: (i, 0, 0)),
            pl.BlockSpec((BN, 9), lambda i: (i, 0)),
            pl.BlockSpec((BN, 3), lambda i: (i, 0)),
            pl.BlockSpec((BN, 1), lambda i: (i, 0)),
            pl.BlockSpec((BN, 1), lambda i: (i, 0)),
            pl.BlockSpec((D, 128), lambda i: (0, 0)),
            pl.BlockSpec((D, 128), lambda i: (0, 0)),
            pl.BlockSpec((D, HEADS * VC), lambda i: (0, 0)),
        ],
        out_specs=[
            pl.BlockSpec((BN, CH), lambda i: (i, 0)),
            pl.BlockSpec((BN, CH), lambda i: (i, 0)),
            pl.BlockSpec((BN, CH), lambda i: (i, 0)),
            pl.BlockSpec((BN, CH), lambda i: (i, 0)),
            pl.BlockSpec((BN, 128), lambda i: (i, 0)),
            pl.BlockSpec((BN, 128), lambda i: (i, 0)),
            pl.BlockSpec((BN, 16), lambda i: (i, 0)),
        ],
        out_shape=[
            fsd((N, CH), jnp.float32), fsd((N, CH), jnp.float32),
            fsd((N, CH), jnp.float32), fsd((N, CH), jnp.float32),
            fsd((N, 128), jnp.float32), fsd((N, 128), jnp.float32),
            fsd((N, 16), jnp.float32),
        ],
    )(nf, rots9, trans, nm, xm, wad, was, wv)


def _k2_body(pd_hbm, ps_hbm, t16_hbm, src_hbm, dst_hbm,
             pdg_hbm, psg_hbm, tsg_hbm, tdg_hbm, dvec_hbm,
             idx_s, idx_d, a, bb, ts, td, dfbuf, sem, semo):
    c = lax.axis_index("c")
    s = lax.axis_index("s")
    wid = c * 16 + s

    def block(j, carry):
        row = wid * NBLK + j
        pltpu.sync_copy(src_hbm.at[row], idx_s)
        pltpu.sync_copy(dst_hbm.at[row], idx_d)
        c1 = pltpu.async_copy(pd_hbm.at[idx_d], a, sem)
        c2 = pltpu.async_copy(ps_hbm.at[idx_s], bb, sem)
        c3 = pltpu.async_copy(t16_hbm.at[idx_s], ts, sem)
        c4 = pltpu.async_copy(t16_hbm.at[idx_d], td, sem)
        for t in range(EB // 16):
            sv = idx_s[pl.ds(t * 16, 16)]
            dv = idx_d[pl.ds(t * 16, 16)]
            dfbuf[pl.ds(t * 16, 16)] = (sv - dv).astype(jnp.float32)
        c1.wait()
        c2.wait()
        c3.wait()
        c4.wait()
        sl = pl.ds(row * EB, EB)
        o1 = pltpu.async_copy(a, pdg_hbm.at[sl], semo)
        o2 = pltpu.async_copy(bb, psg_hbm.at[sl], semo)
        o3 = pltpu.async_copy(ts, tsg_hbm.at[sl], semo)
        o4 = pltpu.async_copy(td, tdg_hbm.at[sl], semo)
        o5 = pltpu.async_copy(dfbuf, dvec_hbm.at[sl], semo)
        o1.wait()
        o2.wait()
        o3.wait()
        o4.wait()
        o5.wait()
        return carry

    lax.fori_loop(0, NBLK, block, 0)


def _k2(pd, ps, t16, src2d, dst2d):
    fsd = jax.ShapeDtypeStruct
    kern = functools.partial(
        pl.kernel,
        out_type=[fsd((EP, 128), jnp.float32), fsd((EP, 128), jnp.float32),
                  fsd((EP, 16), jnp.float32), fsd((EP, 16), jnp.float32),
                  fsd((EP,), jnp.float32)],
        mesh=plsc.VectorSubcoreMesh(core_axis_name="c", subcore_axis_name="s"),
        compiler_params=pltpu.CompilerParams(use_tc_tiling_on_sc=False),
        scratch_types=[
            pltpu.VMEM((EB,), jnp.int32),
            pltpu.VMEM((EB,), jnp.int32),
            pltpu.VMEM((EB, 128), jnp.float32),
            pltpu.VMEM((EB, 128), jnp.float32),
            pltpu.VMEM((EB, 16), jnp.float32),
            pltpu.VMEM((EB, 16), jnp.float32),
            pltpu.VMEM((EB,), jnp.float32),
            pltpu.SemaphoreType.DMA,
            pltpu.SemaphoreType.DMA,
        ],
    )(_k2_body)
    return kern(pd, ps, t16, src2d, dst2d)


def _k3_body(pdg_r, psg_r, tsg_r, tdg_r, dfl_r, wedge, wae, avec,
             mu_r, fr_r, w16):
    psum = pdg_r[...] + psg_r[...]
    b = psum.shape[0]
    vecd = tsg_r[...] - tdg_r[...]
    vec = vecd[:, 0:3]
    dist = jnp.sqrt(jnp.sum(vec * vec, axis=1))
    dfloat = dfl_r[...]
    mu = mu_r[...][0]
    sigma = 20.0 / 16.0
    rbf = jnp.exp(-(((dist[:, None] - mu[None, :]) / sigma) ** 2))
    freq = fr_r[...][0]
    ang = dfloat[:, None] * freq[None, :]
    pe = jnp.concatenate([jnp.cos(ang), jnp.sin(ang)], axis=1)
    ef = jnp.concatenate([rbf, pe], axis=1)
    ee = jax.nn.relu(jnp.dot(ef, wedge[...],
                             preferred_element_type=jnp.float32))
    h = psum[...] + jnp.dot(ee, wae[...],
                            preferred_element_type=jnp.float32)
    h = jax.nn.leaky_relu(h)
    alpha = jnp.dot(h, avec[...],
                    preferred_element_type=jnp.float32)[:, :HEADS]
    valid = jnp.isfinite(dist) & (dist > 1e-3)
    w = jnp.where(valid[:, None], jnp.exp(alpha), 0.0)
    w16[...] = jnp.concatenate([w, jnp.zeros((b, 8), jnp.float32)], axis=1)


def _k3(pdg, psg, tsg, tdg, dvec, wedge, wae, avec):
    nb = EP // BE
    mu = jnp.asarray(_MU, jnp.float32).reshape(1, 16)
    fr = jnp.asarray(_FREQ, jnp.float32).reshape(1, 8)
    return pl.pallas_call(
        _k3_body,
        grid=(nb,),
        in_specs=[
            pl.BlockSpec((BE, 128), lambda i: (i, 0)),
            pl.BlockSpec((BE, 128), lambda i: (i, 0)),
            pl.BlockSpec((BE, 16), lambda i: (i, 0)),
            pl.BlockSpec((BE, 16), lambda i: (i, 0)),
            pl.BlockSpec((BE,), lambda i: (i,)),
            pl.BlockSpec((H, H), lambda i: (0, 0)),
            pl.BlockSpec((H, 128), lambda i: (0, 0)),
            pl.BlockSpec((HEADS * ACH, 16), lambda i: (0, 0)),
            pl.BlockSpec((1, 16), lambda i: (0, 0)),
            pl.BlockSpec((1, 8), lambda i: (0, 0)),
        ],
        out_specs=[pl.BlockSpec((BE, 16), lambda i: (i, 0))],
        out_shape=[jax.ShapeDtypeStruct((EP, 16), jnp.float32)],
    )(pdg, psg, tsg, tdg, dvec, wedge, wae, avec, mu, fr)


def _k4_body(vt0, vt1, vt2, vt3, w16_hbm, src_hbm, dst_hbm, z144, z16,
             a0, a1, a2, a3, den,
             idx_s0, idx_s1, idx_d, vrows0, vrows1, wrows, acc, accd,
             sem0, sem1):
    c = lax.axis_index("c")
    s = lax.axis_index("s")
    wid = c * 16 + s
    rows_per_tile = N // 16
    vts = [vt0, vt1, vt2, vt3]
    outs = [a0, a1, a2, a3]
    lanes = lax.iota(jnp.int32, 16)

    for chunk in range(NCHUNK):
        pltpu.sync_copy(z144.at[pl.ds(s * rows_per_tile, rows_per_tile)],
                        acc.at[pl.ds(s * rows_per_tile, rows_per_tile)])
        if chunk == 0:
            pltpu.sync_copy(z16.at[pl.ds(s * rows_per_tile, rows_per_tile)],
                            accd.at[pl.ds(s * rows_per_tile, rows_per_tile)])
        plsc.subcore_barrier()

        def process(row, vr):
            pltpu.sync_copy(dst_hbm.at[row], idx_d)
            pltpu.sync_copy(w16_hbm.at[pl.ds(row * EB, EB)], wrows)

            def edge(i, carry2):
                wr = wrows[i, :]
                wvs = [jnp.where(lanes < 8, wr[2 * p], wr[2 * p + 1])
                       for p in range(4)]
                for jj in range(CH // 16):
                    p = (9 * chunk + jj) % 4
                    sl = pl.ds(jj * 16, 16)
                    vr[i, sl] = vr[i, sl] * wvs[p]
                return carry2

            lax.fori_loop(0, EB, edge, 0)
            pltpu.sync_copy(vr, acc.at[idx_d], add=True)
            if chunk == 0:
                pltpu.sync_copy(wrows, accd.at[idx_d], add=True)

        row0 = wid * NBLK
        pltpu.sync_copy(src_hbm.at[row0], idx_s0)
        pltpu.async_copy(vts[chunk].at[idx_s0], vrows0, sem0)

        def pair(t, carry):
            rowa = wid * NBLK + 2 * t
            rowb = rowa + 1
            pltpu.sync_copy(src_hbm.at[rowb], idx_s1)
            pltpu.async_copy(vts[chunk].at[idx_s1], vrows1, sem1)
            pltpu.make_async_copy(vts[chunk].at[idx_s0], vrows0, sem0).wait()
            process(rowa, vrows0)

            @pl.when(2 * t + 2 < NBLK)
            def _():
                pltpu.sync_copy(src_hbm.at[rowa + 2], idx_s0)
                pltpu.async_copy(vts[chunk].at[idx_s0], vrows0, sem0)

            pltpu.make_async_copy(vts[chunk].at[idx_s1], vrows1, sem1).wait()
            process(rowb, vrows1)
            return carry

        lax.fori_loop(0, NBLK // 2, pair, 0)
        plsc.subcore_barrier()
        pltpu.sync_copy(
            acc.at[pl.ds(s * rows_per_tile, rows_per_tile)],
            outs[chunk].at[pl.ds(c * N + s * rows_per_tile, rows_per_tile)])
        if chunk == 0:
            pltpu.sync_copy(
                accd.at[pl.ds(s * rows_per_tile, rows_per_tile)],
                den.at[pl.ds(c * N + s * rows_per_tile, rows_per_tile)])
        plsc.subcore_barrier()


def _k4(vt0, vt1, vt2, vt3, w16, src2d, dst2d, z144, z16):
    fsd = jax.ShapeDtypeStruct
    kern = functools.partial(
        pl.kernel,
        out_type=[
            fsd((2 * N, CH), jnp.float32), fsd((2 * N, CH), jnp.float32),
            fsd((2 * N, CH), jnp.float32), fsd((2 * N, CH), jnp.float32),
            fsd((2 * N, 16), jnp.float32),
        ],
        mesh=plsc.VectorSubcoreMesh(core_axis_name="c", subcore_axis_name="s"),
        compiler_params=pltpu.CompilerParams(use_tc_tiling_on_sc=False),
        scratch_types=[
            pltpu.VMEM((EB,), jnp.int32),
            pltpu.VMEM((EB,), jnp.int32),
            pltpu.VMEM((EB,), jnp.int32),
            pltpu.VMEM((EB, CH), jnp.float32),
            pltpu.VMEM((EB, CH), jnp.float32),
            pltpu.VMEM((EB, 16), jnp.float32),
            pltpu.VMEM_SHARED((N, CH), jnp.float32),
            pltpu.VMEM_SHARED((N, 16), jnp.float32),
            pltpu.SemaphoreType.DMA,
            pltpu.SemaphoreType.DMA,
        ],
    )(_k4_body)
    return kern(vt0, vt1, vt2, vt3, w16, src2d, dst2d, z144, z16)


def _k5_body(a0a, a0b, a1a, a1b, a2a, a2b, a3a, a3b, dena, denb,
             rots9, trans, nm, wo, wffn1, wffn2,
             wr1, br1, wr2, br2, wr3, br3, wt,
             rots_out, trans_out, bb_out):
    b = rots9.shape[0]
    den = dena[...] + denb[...]
    den8 = den[:, :HEADS] + 1e-9
    chunks = []
    for (pa, pb, k) in ((a0a, a0b, 0), (a1a, a1b, 1),
                        (a2a, a2b, 2), (a3a, a3b, 3)):
        ck = pa[...] + pb[...]
        dpieces = [jnp.broadcast_to(den8[:, (18 * k + t) % 8][:, None],
                                    (b, 8)) for t in range(18)]
        ck = ck / jnp.concatenate(dpieces, axis=1)
        chunks.append(ck)
    aggf = jnp.concatenate(chunks, axis=1)
    outf = jnp.dot(aggf, wo[...], preferred_element_type=jnp.float32)
    fff = jnp.dot(outf, wffn1[...], preferred_element_type=jnp.float32)
    ff0 = jax.nn.gelu(fff[:, :H])
    fff = jnp.concatenate([ff0, fff[:, H:]], axis=1)
    bbf = outf + jnp.dot(fff, wffn2[...], preferred_element_type=jnp.float32)
    bb_out[...] = bbf
    inv = bbf[:, :H]
    h1 = jax.nn.relu(jnp.dot(inv, wr1[...],
                             preferred_element_type=jnp.float32) + br1[...])
    h2 = jax.nn.relu(jnp.dot(h1, wr2[...],
                             preferred_element_type=jnp.float32) + br2[...])
    qv = jnp.dot(h2, wr3[...], preferred_element_type=jnp.float32) + br3[...]
    qx, qy, qz = qv[:, 0], qv[:, 1], qv[:, 2]
    qn = jnp.sqrt(1.0 + qx * qx + qy * qy + qz * qz)
    w_, x_, y_, z_ = 1.0 / qn, qx / qn, qy / qn, qz / qn
    u = [1 - 2 * (y_ * y_ + z_ * z_), 2 * (x_ * y_ - w_ * z_), 2 * (x_ * z_ + w_ * y_),
         2 * (x_ * y_ + w_ * z_), 1 - 2 * (x_ * x_ + z_ * z_), 2 * (y_ * z_ - w_ * x_),
         2 * (x_ * z_ - w_ * y_), 2 * (y_ * z_ + w_ * x_), 1 - 2 * (x_ * x_ + y_ * y_)]
    tupd = jnp.dot(bbf, wt[...], preferred_element_type=jnp.float32)
    r = rots9[...]
    del rots9
    ru = []
    for i in range(3):
        for kk in range(3):
            ru.append(r[:, 3 * i + 0] * u[0 + kk] + r[:, 3 * i + 1] * u[3 + kk]
                      + r[:, 3 * i + 2] * u[6 + kk])
    rotsu = jnp.stack(ru, axis=1)
    mk = nm[...]
    rots_out[...] = mk * rotsu + (1.0 - mk) * r
    trans_out[...] = trans[...] + mk * tupd


def _k5(a0, a1, a2, a3, den, rots9, trans, nm, wo, wffn1, wffn2,
        wr1, br1, wr2, br2, wr3, br3, wt):
    nb = N // BN
    fsd = jax.ShapeDtypeStruct
    cspec = lambda: pl.BlockSpec((BN, CH), lambda i: (i, 0))
    cspec2 = lambda: pl.BlockSpec((BN, CH), lambda i: (i + N // BN, 0))
    full = lambda shape: pl.BlockSpec(shape, lambda i: tuple(0 for _ in shape))
    return pl.pallas_call(
        _k5_body,
        grid=(nb,),
        in_specs=[
            cspec(), cspec2(), cspec(), cspec2(),
            cspec(), cspec2(), cspec(), cspec2(),
            pl.BlockSpec((BN, 16), lambda i: (i, 0)),
            pl.BlockSpec((BN, 16), lambda i: (i + N // BN, 0)),
            pl.BlockSpec((BN, 9), lambda i: (i, 0)),
            pl.BlockSpec((BN, 3), lambda i: (i, 0)),
            pl.BlockSpec((BN, 1), lambda i: (i, 0)),
            full((NCHUNK * CH, SPH * H)), full((SPH * H, SPH * H)),
            full((SPH * H, SPH * H)),
            full((H, 2 * H)), full((2 * H,)), full((2 * H, H)), full((H,)),
            full((H, 6)), full((6,)), full((SPH * H, 3)),
        ],
        out_specs=[
            pl.BlockSpec((BN, 9), lambda i: (i, 0)),
            pl.BlockSpec((BN, 3), lambda i: (i, 0)),
            pl.BlockSpec((BN, SPH * H), lambda i: (i, 0)),
        ],
        out_shape=[
            fsd((N, 9), jnp.float32),
            fsd((N, 3), jnp.float32),
            fsd((N, SPH * H), jnp.float32),
        ],
    )(a0, a0, a1, a1, a2, a2, a3, a3, den, den, rots9, trans, nm,
      wo, wffn1, wffn2, wr1, br1, wr2, br2, wr3, br3, wt)


def kernel(rots, trans, node_features, batch, x_mask, noising_mask,
           sampled_edge_index, seq_local_edge_index, W_edge, W_alpha, a_vec,
           W_v, W_o, W_ffn1, W_ffn2, W_rot1, b_rot1, W_rot2, b_rot2,
           W_rot3, b_rot3, W_t):
    del batch
    ei = jnp.concatenate([sampled_edge_index, seq_local_edge_index], axis=-1)
    src = jnp.pad(ei[0], (0, EP - E)).reshape(NROWS, EB)
    dst = jnp.pad(ei[1], (0, EP - E)).reshape(NROWS, EB)
    rots9 = rots.reshape(N, 9)
    nm = noising_mask.astype(jnp.float32).reshape(N, 1)
    xm = x_mask.astype(jnp.float32).reshape(N, 1)
    wad = W_alpha[:D]
    was = W_alpha[D:2 * D]
    wae = W_alpha[2 * D:]
    a16 = jnp.zeros((HEADS * ACH, 16), jnp.float32).at[
        jnp.arange(HEADS * ACH), jnp.arange(HEADS * ACH) // ACH
    ].set(a_vec.reshape(-1))
    eye9 = jnp.eye(SPH, dtype=jnp.float32)
    wo_bd = jnp.kron(eye9, W_o)
    wffn1_bd = jnp.kron(eye9, W_ffn1)
    wffn2_bd = jnp.kron(eye9, W_ffn2)
    p93 = jnp.zeros((SPH, 3), jnp.float32).at[jnp.arange(1, 4),
                                              jnp.arange(3)].set(1.0)
    wt_bd = jnp.kron(p93, W_t)
    z144 = jnp.zeros((N, CH), jnp.float32)
    z16 = jnp.zeros((N, 16), jnp.float32)

    vt0, vt1, vt2, vt3, pd, ps, t16 = _k1(
        node_features, rots9, trans, nm, xm, wad, was, W_v)
    pdg, psg, tsg, tdg, dvec = _k2(pd, ps, t16, src, dst)
    (w16,) = _k3(pdg, psg, tsg, tdg, dvec, W_edge, wae, a16)
    a0, a1, a2, a3, den = _k4(vt0, vt1, vt2, vt3, w16, src, dst, z144, z16)
    rots9f, transf, bb = _k5(a0, a1, a2, a3, den, rots9, trans, nm,
                             wo_bd, wffn1_bd, wffn2_bd, W_rot1, b_rot1,
                             W_rot2, b_rot2, W_rot3, b_rot3, wt_bd)
    return rots9f.reshape(N, 3, 3), transf, bb.reshape(N, SPH, H)
